# baseline (speedup 1.0000x reference)
import functools

import jax
import jax.numpy as jnp
from jax import lax
from jax.experimental import pallas as pl
from jax.experimental.pallas import tpu as pltpu

N_TOK = 2048
D_IN = 2048
D_MKT = 16
N_EXP = 8
TOP_K = 2
D_H = 1024
D_OUT = 512

TB = 1024
N_TB = N_TOK // TB

BLK = 256
NB = N_TOK * TOP_K // BLK + N_EXP
NSLOT = NB * BLK

CB = 256
N_CB = N_TOK // CB

_INV_SQRT2 = 0.7071067811865476


def _gate_body(x_ref, mkt_ref, wgx_ref, wgm_ref, bg_ref,
               probs_ref, eidx_ref, ew_ref):
    logits = (jnp.dot(x_ref[...], wgx_ref[...], preferred_element_type=jnp.float32)
              + jnp.dot(mkt_ref[...], wgm_ref[...], preferred_element_type=jnp.float32)
              + bg_ref[...])
    z = logits - jnp.max(logits, axis=-1, keepdims=True)
    p = jnp.exp(z)
    p = p / jnp.sum(p, axis=-1, keepdims=True)
    probs_ref[...] = p

    iota = lax.broadcasted_iota(jnp.int32, p.shape, 1)
    m1 = jnp.max(p, axis=-1, keepdims=True)
    idx1 = jnp.min(jnp.where(p == m1, iota, N_EXP), axis=-1, keepdims=True)
    p2 = jnp.where(iota == idx1, -jnp.inf, p)
    m2 = jnp.max(p2, axis=-1, keepdims=True)
    idx2 = jnp.min(jnp.where(p2 == m2, iota, N_EXP), axis=-1, keepdims=True)
    denom = m1 + m2 + 1e-8
    eidx_ref[...] = jnp.concatenate([idx1, idx2], axis=1)
    ew_ref[...] = jnp.concatenate([m1 / denom, m2 / denom], axis=1)


def _route_body(eidx_ref, slot0_ref, slot1_ref, bexp_ref, nused_ref):
    eidx = eidx_ref[...]
    iota_e = lax.broadcasted_iota(jnp.int32, (N_TOK, N_EXP), 1)
    m0 = (eidx[:, 0:1] == iota_e).astype(jnp.float32)
    m1 = (eidx[:, 1:2] == iota_e).astype(jnp.float32)
    s = m0 + m1

    ti = lax.broadcasted_iota(jnp.int32, (CB, CB), 0)
    tj = lax.broadcasted_iota(jnp.int32, (CB, CB), 1)
    tri = (tj < ti).astype(jnp.float32)
    blocks = []
    carry = jnp.zeros((1, N_EXP), jnp.float32)
    for b in range(N_CB):
        sb = s[b * CB:(b + 1) * CB, :]
        blocks.append(jnp.dot(tri, sb, preferred_element_type=jnp.float32) + carry)
        carry = carry + jnp.sum(sb, axis=0, keepdims=True)
    ec = jnp.concatenate(blocks, axis=0)
    counts = carry

    nb = jnp.floor((counts + (BLK - 1)) * (1.0 / BLK))
    upper = (lax.broadcasted_iota(jnp.int32, (N_EXP, N_EXP), 0)
             < lax.broadcasted_iota(jnp.int32, (N_EXP, N_EXP), 1)).astype(jnp.float32)
    startb = jnp.dot(nb, upper, preferred_element_type=jnp.float32)
    start_rows = startb * float(BLK)

    slot0 = jnp.sum(m0 * (ec + start_rows), axis=1, keepdims=True)
    slot1 = jnp.sum(m1 * (ec + m0 + start_rows), axis=1, keepdims=True)
    slot0_ref[...] = slot0.astype(jnp.int32)
    slot1_ref[...] = slot1.astype(jnp.int32)

    iota_b = lax.broadcasted_iota(jnp.int32, (NB, N_EXP), 0).astype(jnp.float32)
    cmp = (startb <= iota_b + 0.5).astype(jnp.float32)
    bexp_ref[...] = (jnp.sum(cmp, axis=1, keepdims=True) - 1.0).astype(jnp.int32)
    nused_ref[...] = jnp.sum(nb, axis=1, keepdims=True).astype(jnp.int32)


def _ffn_body(bexp_ref, nused_ref, xs_ref, w1_ref, b1_ref, w2_ref, b2_ref, y_ref):
    b = pl.program_id(0)

    @pl.when(b < nused_ref[0])
    def _():
        h = jnp.dot(xs_ref[...], w1_ref[0], preferred_element_type=jnp.float32) + b1_ref[0]
        h = 0.5 * h * (1.0 + lax.erf(h * _INV_SQRT2))
        y_ref[...] = jnp.dot(h, w2_ref[0], preferred_element_type=jnp.float32) + b2_ref[0]


@jax.jit
def kernel(x, market_status, W_g, b_g, W1, b1, W2, b2):
    wgx = W_g[:D_IN]
    wgm = W_g[D_IN:]
    bg2 = b_g.reshape(1, N_EXP)

    probs, eidx, ew = pl.pallas_call(
        _gate_body,
        grid=(N_TB,),
        in_specs=[
            pl.BlockSpec((TB, D_IN), lambda t: (t, 0)),
            pl.BlockSpec((TB, D_MKT), lambda t: (t, 0)),
            pl.BlockSpec((D_IN, N_EXP), lambda t: (0, 0)),
            pl.BlockSpec((D_MKT, N_EXP), lambda t: (0, 0)),
            pl.BlockSpec((1, N_EXP), lambda t: (0, 0)),
        ],
        out_specs=[
            pl.BlockSpec((TB, N_EXP), lambda t: (t, 0)),
            pl.BlockSpec((TB, TOP_K), lambda t: (t, 0)),
            pl.BlockSpec((TB, TOP_K), lambda t: (t, 0)),
        ],
        out_shape=[
            jax.ShapeDtypeStruct((N_TOK, N_EXP), jnp.float32),
            jax.ShapeDtypeStruct((N_TOK, TOP_K), jnp.int32),
            jax.ShapeDtypeStruct((N_TOK, TOP_K), jnp.float32),
        ],
    )(x, market_status, wgx, wgm, bg2)

    slot0, slot1, bexp, nused = pl.pallas_call(
        _route_body,
        out_shape=[
            jax.ShapeDtypeStruct((N_TOK, 1), jnp.int32),
            jax.ShapeDtypeStruct((N_TOK, 1), jnp.int32),
            jax.ShapeDtypeStruct((NB, 1), jnp.int32),
            jax.ShapeDtypeStruct((1, 1), jnp.int32),
        ],
    )(eidx)

    s0 = slot0.reshape(N_TOK)
    s1 = slot1.reshape(N_TOK)

    xs = jnp.zeros((NSLOT, D_IN), jnp.float32).at[s0].set(x).at[s1].set(x)

    grid_spec = pltpu.PrefetchScalarGridSpec(
        num_scalar_prefetch=2,
        grid=(NB,),
        in_specs=[
            pl.BlockSpec((BLK, D_IN), lambda b, be, nu: (b, 0)),
            pl.BlockSpec((1, D_IN, D_H), lambda b, be, nu: (be[b], 0, 0)),
            pl.BlockSpec((1, 1, D_H), lambda b, be, nu: (be[b], 0, 0)),
            pl.BlockSpec((1, D_H, D_OUT), lambda b, be, nu: (be[b], 0, 0)),
            pl.BlockSpec((1, 1, D_OUT), lambda b, be, nu: (be[b], 0, 0)),
        ],
        out_specs=pl.BlockSpec((BLK, D_OUT), lambda b, be, nu: (b, 0)),
    )
    y = pl.pallas_call(
        _ffn_body,
        grid_spec=grid_spec,
        out_shape=jax.ShapeDtypeStruct((NSLOT, D_OUT), jnp.float32),
    )(bexp.reshape(NB), nused.reshape(1), xs, W1,
      b1.reshape(N_EXP, 1, D_H), W2, b2.reshape(N_EXP, 1, D_OUT))

    out = ew[:, 0:1] * y[s0] + ew[:, 1:2] * y[s1]
    return out, probs


# speedup vs baseline: 1.4362x; 1.4362x over previous
import functools

import jax
import jax.numpy as jnp
from jax import lax
from jax.experimental import pallas as pl
from jax.experimental.pallas import tpu as pltpu
from jax.experimental.pallas import tpu_sc as plsc

N_TOK = 2048
D_IN = 2048
D_MKT = 16
N_EXP = 8
TOP_K = 2
D_H = 1024
D_OUT = 512

TB = 1024
N_TB = N_TOK // TB

BLK = 256
NB = N_TOK * TOP_K // BLK + N_EXP
NSLOT = NB * BLK

CB = 256
N_CB = N_TOK // CB

_INV_SQRT2 = 0.7071067811865476


def _gate_body(x_ref, mkt_ref, wgx_ref, wgm_ref, bg_ref,
               probs_ref, eidx_ref, ew_ref):
    logits = (jnp.dot(x_ref[...], wgx_ref[...], preferred_element_type=jnp.float32)
              + jnp.dot(mkt_ref[...], wgm_ref[...], preferred_element_type=jnp.float32)
              + bg_ref[...])
    z = logits - jnp.max(logits, axis=-1, keepdims=True)
    p = jnp.exp(z)
    p = p / jnp.sum(p, axis=-1, keepdims=True)
    probs_ref[...] = p

    iota = lax.broadcasted_iota(jnp.int32, p.shape, 1)
    m1 = jnp.max(p, axis=-1, keepdims=True)
    idx1 = jnp.min(jnp.where(p == m1, iota, N_EXP), axis=-1, keepdims=True)
    p2 = jnp.where(iota == idx1, -jnp.inf, p)
    m2 = jnp.max(p2, axis=-1, keepdims=True)
    idx2 = jnp.min(jnp.where(p2 == m2, iota, N_EXP), axis=-1, keepdims=True)
    denom = m1 + m2 + 1e-8
    eidx_ref[...] = jnp.concatenate([idx1, idx2], axis=1)
    ew_ref[...] = jnp.concatenate([m1 / denom, m2 / denom], axis=1)


def _route_body(eidx_ref, slot0_ref, slot1_ref, bexp_ref, nused_ref):
    eidx = eidx_ref[...]
    iota_e = lax.broadcasted_iota(jnp.int32, (N_TOK, N_EXP), 1)
    m0 = (eidx[:, 0:1] == iota_e).astype(jnp.float32)
    m1 = (eidx[:, 1:2] == iota_e).astype(jnp.float32)
    s = m0 + m1

    ti = lax.broadcasted_iota(jnp.int32, (CB, CB), 0)
    tj = lax.broadcasted_iota(jnp.int32, (CB, CB), 1)
    tri = (tj < ti).astype(jnp.float32)
    blocks = []
    carry = jnp.zeros((1, N_EXP), jnp.float32)
    for b in range(N_CB):
        sb = s[b * CB:(b + 1) * CB, :]
        blocks.append(jnp.dot(tri, sb, preferred_element_type=jnp.float32) + carry)
        carry = carry + jnp.sum(sb, axis=0, keepdims=True)
    ec = jnp.concatenate(blocks, axis=0)
    counts = carry

    nb = jnp.floor((counts + (BLK - 1)) * (1.0 / BLK))
    upper = (lax.broadcasted_iota(jnp.int32, (N_EXP, N_EXP), 0)
             < lax.broadcasted_iota(jnp.int32, (N_EXP, N_EXP), 1)).astype(jnp.float32)
    startb = jnp.dot(nb, upper, preferred_element_type=jnp.float32)
    start_rows = startb * float(BLK)

    slot0 = jnp.sum(m0 * (ec + start_rows), axis=1, keepdims=True)
    slot1 = jnp.sum(m1 * (ec + m0 + start_rows), axis=1, keepdims=True)
    slot0_ref[...] = slot0.astype(jnp.int32)
    slot1_ref[...] = slot1.astype(jnp.int32)

    iota_b = lax.broadcasted_iota(jnp.int32, (NB, N_EXP), 0).astype(jnp.float32)
    cmp = (startb <= iota_b + 0.5).astype(jnp.float32)
    bexp_ref[...] = (jnp.sum(cmp, axis=1, keepdims=True) - 1.0).astype(jnp.int32)
    nused_ref[...] = jnp.sum(nb, axis=1, keepdims=True).astype(jnp.int32)


N_WORKERS = 32
TOK_PER_W = N_TOK // N_WORKERS
SC_CHUNK = 32
N_CHUNKS = TOK_PER_W // SC_CHUNK


def _scatter_sc_body(x_hbm, slots_hbm, xs_hbm, xbuf, idxbuf, sem):
    wid = lax.axis_index("s") * 2 + lax.axis_index("c")
    for c in range(N_CHUNKS):
        j = wid * N_CHUNKS + c
        base = j * SC_CHUNK
        pltpu.sync_copy(x_hbm.at[pl.ds(base, SC_CHUNK)], xbuf)
        pltpu.sync_copy(slots_hbm.at[j], idxbuf)
        pltpu.async_copy(xbuf, xs_hbm.at[idxbuf.at[0]], sem).wait()
        pltpu.async_copy(xbuf, xs_hbm.at[idxbuf.at[1]], sem).wait()


_scatter_sc = functools.partial(
    pl.kernel,
    mesh=plsc.VectorSubcoreMesh(core_axis_name="c", subcore_axis_name="s"),
    out_type=jax.ShapeDtypeStruct((NSLOT, D_IN), jnp.float32),
    scratch_types=[
        pltpu.VMEM((SC_CHUNK, D_IN), jnp.float32),
        pltpu.VMEM((2, SC_CHUNK), jnp.int32),
        pltpu.SemaphoreType.DMA,
    ],
)(_scatter_sc_body)


def _ffn_body(bexp_ref, nused_ref, xs_ref, w1_ref, b1_ref, w2_ref, b2_ref, y_ref):
    b = pl.program_id(0)

    @pl.when(b < nused_ref[0])
    def _():
        h = jnp.dot(xs_ref[...], w1_ref[0], preferred_element_type=jnp.float32) + b1_ref[0]
        h = 0.5 * h * (1.0 + lax.erf(h * _INV_SQRT2))
        y_ref[...] = jnp.dot(h, w2_ref[0], preferred_element_type=jnp.float32) + b2_ref[0]


@jax.jit
def kernel(x, market_status, W_g, b_g, W1, b1, W2, b2):
    wgx = W_g[:D_IN]
    wgm = W_g[D_IN:]
    bg2 = b_g.reshape(1, N_EXP)

    probs, eidx, ew = pl.pallas_call(
        _gate_body,
        grid=(N_TB,),
        in_specs=[
            pl.BlockSpec((TB, D_IN), lambda t: (t, 0)),
            pl.BlockSpec((TB, D_MKT), lambda t: (t, 0)),
            pl.BlockSpec((D_IN, N_EXP), lambda t: (0, 0)),
            pl.BlockSpec((D_MKT, N_EXP), lambda t: (0, 0)),
            pl.BlockSpec((1, N_EXP), lambda t: (0, 0)),
        ],
        out_specs=[
            pl.BlockSpec((TB, N_EXP), lambda t: (t, 0)),
            pl.BlockSpec((TB, TOP_K), lambda t: (t, 0)),
            pl.BlockSpec((TB, TOP_K), lambda t: (t, 0)),
        ],
        out_shape=[
            jax.ShapeDtypeStruct((N_TOK, N_EXP), jnp.float32),
            jax.ShapeDtypeStruct((N_TOK, TOP_K), jnp.int32),
            jax.ShapeDtypeStruct((N_TOK, TOP_K), jnp.float32),
        ],
    )(x, market_status, wgx, wgm, bg2)

    slot0, slot1, bexp, nused = pl.pallas_call(
        _route_body,
        out_shape=[
            jax.ShapeDtypeStruct((N_TOK, 1), jnp.int32),
            jax.ShapeDtypeStruct((N_TOK, 1), jnp.int32),
            jax.ShapeDtypeStruct((NB, 1), jnp.int32),
            jax.ShapeDtypeStruct((1, 1), jnp.int32),
        ],
    )(eidx)

    s0 = slot0.reshape(N_TOK)
    s1 = slot1.reshape(N_TOK)

    slots_sc = jnp.stack(
        [slot0.reshape(N_WORKERS * N_CHUNKS, SC_CHUNK),
         slot1.reshape(N_WORKERS * N_CHUNKS, SC_CHUNK)], axis=1)
    xs = _scatter_sc(x, slots_sc)

    grid_spec = pltpu.PrefetchScalarGridSpec(
        num_scalar_prefetch=2,
        grid=(NB,),
        in_specs=[
            pl.BlockSpec((BLK, D_IN), lambda b, be, nu: (b, 0)),
            pl.BlockSpec((1, D_IN, D_H), lambda b, be, nu: (be[b], 0, 0)),
            pl.BlockSpec((1, 1, D_H), lambda b, be, nu: (be[b], 0, 0)),
            pl.BlockSpec((1, D_H, D_OUT), lambda b, be, nu: (be[b], 0, 0)),
            pl.BlockSpec((1, 1, D_OUT), lambda b, be, nu: (be[b], 0, 0)),
        ],
        out_specs=pl.BlockSpec((BLK, D_OUT), lambda b, be, nu: (b, 0)),
    )
    y = pl.pallas_call(
        _ffn_body,
        grid_spec=grid_spec,
        out_shape=jax.ShapeDtypeStruct((NSLOT, D_OUT), jnp.float32),
    )(bexp.reshape(NB), nused.reshape(1), xs, W1,
      b1.reshape(N_EXP, 1, D_H), W2, b2.reshape(N_EXP, 1, D_OUT))

    out = ew[:, 0:1] * y[s0] + ew[:, 1:2] * y[s1]
    return out, probs


# speedup vs baseline: 1.4691x; 1.0229x over previous
import functools

import jax
import jax.numpy as jnp
from jax import lax
from jax.experimental import pallas as pl
from jax.experimental.pallas import tpu as pltpu
from jax.experimental.pallas import tpu_sc as plsc

N_TOK = 2048
D_IN = 2048
D_MKT = 16
N_EXP = 8
TOP_K = 2
D_H = 1024
D_OUT = 512

BLK = 256
NB = N_TOK * TOP_K // BLK + N_EXP
NSLOT = NB * BLK

CB = 256
N_CB = N_TOK // CB

N_WORKERS = 32
SC_CHUNK = 32
N_CHUNKS = N_TOK // (N_WORKERS * SC_CHUNK)

_INV_SQRT2 = 0.7071067811865476


def _gate_route_body(x_ref, mkt_ref, wgx_ref, wgm_ref, bg_ref,
                     probs_ref, slot0_ref, slot1_ref, ew_ref,
                     bexp_ref, nused_ref):
    logits = (jnp.dot(x_ref[...], wgx_ref[...], preferred_element_type=jnp.float32)
              + jnp.dot(mkt_ref[...], wgm_ref[...], preferred_element_type=jnp.float32)
              + bg_ref[...])
    z = logits - jnp.max(logits, axis=-1, keepdims=True)
    p = jnp.exp(z)
    p = p / jnp.sum(p, axis=-1, keepdims=True)
    probs_ref[...] = p

    iota = lax.broadcasted_iota(jnp.int32, p.shape, 1)
    mx1 = jnp.max(p, axis=-1, keepdims=True)
    idx1 = jnp.min(jnp.where(p == mx1, iota, N_EXP), axis=-1, keepdims=True)
    p2 = jnp.where(iota == idx1, -jnp.inf, p)
    mx2 = jnp.max(p2, axis=-1, keepdims=True)
    idx2 = jnp.min(jnp.where(p2 == mx2, iota, N_EXP), axis=-1, keepdims=True)
    denom = mx1 + mx2 + 1e-8
    ew_ref[...] = jnp.concatenate([mx1 / denom, mx2 / denom], axis=1)

    m0 = (idx1 == iota).astype(jnp.float32)
    m1 = (idx2 == iota).astype(jnp.float32)
    s = m0 + m1

    ti = lax.broadcasted_iota(jnp.int32, (CB, CB), 0)
    tj = lax.broadcasted_iota(jnp.int32, (CB, CB), 1)
    tri = (tj < ti).astype(jnp.float32)
    blocks = []
    carry = jnp.zeros((1, N_EXP), jnp.float32)
    for b in range(N_CB):
        sb = s[b * CB:(b + 1) * CB, :]
        blocks.append(jnp.dot(tri, sb, preferred_element_type=jnp.float32) + carry)
        carry = carry + jnp.sum(sb, axis=0, keepdims=True)
    ec = jnp.concatenate(blocks, axis=0)
    counts = carry

    nb = jnp.floor((counts + (BLK - 1)) * (1.0 / BLK))
    upper = (lax.broadcasted_iota(jnp.int32, (N_EXP, N_EXP), 0)
             < lax.broadcasted_iota(jnp.int32, (N_EXP, N_EXP), 1)).astype(jnp.float32)
    startb = jnp.dot(nb, upper, preferred_element_type=jnp.float32)
    start_rows = startb * float(BLK)

    slot0 = jnp.sum(m0 * (ec + start_rows), axis=1, keepdims=True)
    slot1 = jnp.sum(m1 * (ec + m0 + start_rows), axis=1, keepdims=True)
    slot0_ref[...] = slot0.astype(jnp.int32)
    slot1_ref[...] = slot1.astype(jnp.int32)

    iota_b = lax.broadcasted_iota(jnp.int32, (NB, N_EXP), 0).astype(jnp.float32)
    cmp = (startb <= iota_b + 0.5).astype(jnp.float32)
    bexp_ref[...] = (jnp.sum(cmp, axis=1, keepdims=True) - 1.0).astype(jnp.int32)
    nused_ref[...] = jnp.sum(nb, axis=1, keepdims=True).astype(jnp.int32)


def _scatter_sc_body(x_hbm, slots_hbm, xs_hbm, xbuf, idxbuf, sem):
    wid = lax.axis_index("s") * 2 + lax.axis_index("c")
    for c in range(N_CHUNKS):
        j = wid * N_CHUNKS + c
        base = j * SC_CHUNK
        pltpu.sync_copy(x_hbm.at[pl.ds(base, SC_CHUNK)], xbuf)
        pltpu.sync_copy(slots_hbm.at[j], idxbuf)
        c0 = pltpu.async_copy(xbuf, xs_hbm.at[idxbuf.at[0]], sem)
        c1 = pltpu.async_copy(xbuf, xs_hbm.at[idxbuf.at[1]], sem)
        c0.wait()
        c1.wait()


_scatter_sc = functools.partial(
    pl.kernel,
    mesh=plsc.VectorSubcoreMesh(core_axis_name="c", subcore_axis_name="s"),
    out_type=jax.ShapeDtypeStruct((NSLOT, D_IN), jnp.float32),
    scratch_types=[
        pltpu.VMEM((SC_CHUNK, D_IN), jnp.float32),
        pltpu.VMEM((2, SC_CHUNK), jnp.int32),
        pltpu.SemaphoreType.DMA,
    ],
)(_scatter_sc_body)


def _ffn_body(bexp_ref, nused_ref, xs_ref, w1_ref, b1_ref, w2_ref, b2_ref, y_ref):
    b = pl.program_id(0)

    @pl.when(b < nused_ref[0])
    def _():
        h = jnp.dot(xs_ref[...], w1_ref[0], preferred_element_type=jnp.float32) + b1_ref[0]
        h = 0.5 * h * (1.0 + lax.erf(h * _INV_SQRT2))
        y_ref[...] = jnp.dot(h, w2_ref[0], preferred_element_type=jnp.float32) + b2_ref[0]


@jax.jit
def kernel(x, market_status, W_g, b_g, W1, b1, W2, b2):
    wgx = W_g[:D_IN]
    wgm = W_g[D_IN:]
    bg2 = b_g.reshape(1, N_EXP)

    full = lambda shape: pl.BlockSpec(shape, lambda: tuple(0 for _ in shape))
    probs, slot0, slot1, ew, bexp, nused = pl.pallas_call(
        _gate_route_body,
        in_specs=[
            full((N_TOK, D_IN)),
            full((N_TOK, D_MKT)),
            full((D_IN, N_EXP)),
            full((D_MKT, N_EXP)),
            full((1, N_EXP)),
        ],
        out_specs=[
            full((N_TOK, N_EXP)),
            full((N_TOK, 1)),
            full((N_TOK, 1)),
            full((N_TOK, TOP_K)),
            full((NB, 1)),
            full((1, 1)),
        ],
        out_shape=[
            jax.ShapeDtypeStruct((N_TOK, N_EXP), jnp.float32),
            jax.ShapeDtypeStruct((N_TOK, 1), jnp.int32),
            jax.ShapeDtypeStruct((N_TOK, 1), jnp.int32),
            jax.ShapeDtypeStruct((N_TOK, TOP_K), jnp.float32),
            jax.ShapeDtypeStruct((NB, 1), jnp.int32),
            jax.ShapeDtypeStruct((1, 1), jnp.int32),
        ],
    )(x, market_status, wgx, wgm, bg2)

    nrows = N_WORKERS * N_CHUNKS
    slots_sc = jnp.stack(
        [slot0.reshape(nrows, SC_CHUNK), slot1.reshape(nrows, SC_CHUNK)], axis=1)
    ws_sc = jnp.stack(
        [ew[:, 0].reshape(nrows, SC_CHUNK), ew[:, 1].reshape(nrows, SC_CHUNK)],
        axis=1)

    xs = _scatter_sc(x, slots_sc)

    grid_spec = pltpu.PrefetchScalarGridSpec(
        num_scalar_prefetch=2,
        grid=(NB,),
        in_specs=[
            pl.BlockSpec((BLK, D_IN),
                         lambda b, be, nu: (jnp.minimum(b, nu[0] - 1), 0)),
            pl.BlockSpec((1, D_IN, D_H), lambda b, be, nu: (be[b], 0, 0)),
            pl.BlockSpec((1, 1, D_H), lambda b, be, nu: (be[b], 0, 0)),
            pl.BlockSpec((1, D_H, D_OUT), lambda b, be, nu: (be[b], 0, 0)),
            pl.BlockSpec((1, 1, D_OUT), lambda b, be, nu: (be[b], 0, 0)),
        ],
        out_specs=pl.BlockSpec((BLK, D_OUT), lambda b, be, nu: (b, 0)),
    )
    y = pl.pallas_call(
        _ffn_body,
        grid_spec=grid_spec,
        out_shape=jax.ShapeDtypeStruct((NSLOT, D_OUT), jnp.float32),
    )(bexp.reshape(NB), nused.reshape(1), xs, W1,
      b1.reshape(N_EXP, 1, D_H), W2, b2.reshape(N_EXP, 1, D_OUT))

    s0 = slot0.reshape(N_TOK)
    s1 = slot1.reshape(N_TOK)
    out = ew[:, 0:1] * y[s0] + ew[:, 1:2] * y[s1]
    return out, probs


# speedup vs baseline: 1.5174x; 1.0329x over previous
import functools

import jax
import jax.numpy as jnp
from jax import lax
from jax.experimental import pallas as pl
from jax.experimental.pallas import tpu as pltpu
from jax.experimental.pallas import tpu_sc as plsc

N_TOK = 2048
D_IN = 2048
D_MKT = 16
N_EXP = 8
TOP_K = 2
D_H = 1024
D_OUT = 512

BLK = 512
NB = N_TOK * TOP_K // BLK + N_EXP
NSLOT = NB * BLK

CB = 256
N_CB = N_TOK // CB

N_WORKERS = 32
SC_CHUNK = 32
N_CHUNKS = N_TOK // (N_WORKERS * SC_CHUNK)

_INV_SQRT2 = 0.7071067811865476


def _gate_route_body(x_ref, mkt_ref, wgx_ref, wgm_ref, bg_ref,
                     probs_ref, slot0_ref, slot1_ref, ew_ref,
                     bexp_ref, nused_ref):
    logits = (jnp.dot(x_ref[...], wgx_ref[...], preferred_element_type=jnp.float32)
              + jnp.dot(mkt_ref[...], wgm_ref[...], preferred_element_type=jnp.float32)
              + bg_ref[...])
    z = logits - jnp.max(logits, axis=-1, keepdims=True)
    p = jnp.exp(z)
    p = p / jnp.sum(p, axis=-1, keepdims=True)
    probs_ref[...] = p

    iota = lax.broadcasted_iota(jnp.int32, p.shape, 1)
    mx1 = jnp.max(p, axis=-1, keepdims=True)
    idx1 = jnp.min(jnp.where(p == mx1, iota, N_EXP), axis=-1, keepdims=True)
    p2 = jnp.where(iota == idx1, -jnp.inf, p)
    mx2 = jnp.max(p2, axis=-1, keepdims=True)
    idx2 = jnp.min(jnp.where(p2 == mx2, iota, N_EXP), axis=-1, keepdims=True)
    denom = mx1 + mx2 + 1e-8
    ew_ref[...] = jnp.concatenate([mx1 / denom, mx2 / denom], axis=1)

    m0 = (idx1 == iota).astype(jnp.float32)
    m1 = (idx2 == iota).astype(jnp.float32)
    s = m0 + m1

    ti = lax.broadcasted_iota(jnp.int32, (CB, CB), 0)
    tj = lax.broadcasted_iota(jnp.int32, (CB, CB), 1)
    tri = (tj < ti).astype(jnp.float32)
    blocks = []
    carry = jnp.zeros((1, N_EXP), jnp.float32)
    for b in range(N_CB):
        sb = s[b * CB:(b + 1) * CB, :]
        blocks.append(jnp.dot(tri, sb, preferred_element_type=jnp.float32) + carry)
        carry = carry + jnp.sum(sb, axis=0, keepdims=True)
    ec = jnp.concatenate(blocks, axis=0)
    counts = carry

    nb = jnp.floor((counts + (BLK - 1)) * (1.0 / BLK))
    upper = (lax.broadcasted_iota(jnp.int32, (N_EXP, N_EXP), 0)
             < lax.broadcasted_iota(jnp.int32, (N_EXP, N_EXP), 1)).astype(jnp.float32)
    startb = jnp.dot(nb, upper, preferred_element_type=jnp.float32)
    start_rows = startb * float(BLK)

    slot0 = jnp.sum(m0 * (ec + start_rows), axis=1, keepdims=True)
    slot1 = jnp.sum(m1 * (ec + m0 + start_rows), axis=1, keepdims=True)
    slot0_ref[...] = slot0.astype(jnp.int32)
    slot1_ref[...] = slot1.astype(jnp.int32)

    iota_b = lax.broadcasted_iota(jnp.int32, (NB, N_EXP), 0).astype(jnp.float32)
    cmp = (startb <= iota_b + 0.5).astype(jnp.float32)
    bexp_ref[...] = (jnp.sum(cmp, axis=1, keepdims=True) - 1.0).astype(jnp.int32)
    nused_ref[...] = jnp.sum(nb, axis=1, keepdims=True).astype(jnp.int32)


def _scatter_sc_body(x_hbm, slots_hbm, xs_hbm, xbuf, idxbuf, sem):
    wid = lax.axis_index("s") * 2 + lax.axis_index("c")
    for c in range(N_CHUNKS):
        j = wid * N_CHUNKS + c
        base = j * SC_CHUNK
        pltpu.sync_copy(x_hbm.at[pl.ds(base, SC_CHUNK)], xbuf)
        pltpu.sync_copy(slots_hbm.at[j], idxbuf)
        c0 = pltpu.async_copy(xbuf, xs_hbm.at[idxbuf.at[0]], sem)
        c1 = pltpu.async_copy(xbuf, xs_hbm.at[idxbuf.at[1]], sem)
        c0.wait()
        c1.wait()


_scatter_sc = functools.partial(
    pl.kernel,
    mesh=plsc.VectorSubcoreMesh(core_axis_name="c", subcore_axis_name="s"),
    out_type=jax.ShapeDtypeStruct((NSLOT, D_IN), jnp.float32),
    scratch_types=[
        pltpu.VMEM((SC_CHUNK, D_IN), jnp.float32),
        pltpu.VMEM((2, SC_CHUNK), jnp.int32),
        pltpu.SemaphoreType.DMA,
    ],
)(_scatter_sc_body)


def _ffn_body(bexp_ref, nused_ref, xs_ref, w1_ref, b1_ref, w2_ref, b2_ref, y_ref):
    b = pl.program_id(0)

    @pl.when(b < nused_ref[0])
    def _():
        h = jnp.dot(xs_ref[...], w1_ref[0], preferred_element_type=jnp.float32) + b1_ref[0]
        h = 0.5 * h * (1.0 + lax.erf(h * _INV_SQRT2))
        y_ref[...] = jnp.dot(h, w2_ref[0], preferred_element_type=jnp.float32) + b2_ref[0]


@jax.jit
def kernel(x, market_status, W_g, b_g, W1, b1, W2, b2):
    wgx = W_g[:D_IN]
    wgm = W_g[D_IN:]
    bg2 = b_g.reshape(1, N_EXP)

    full = lambda shape: pl.BlockSpec(shape, lambda: tuple(0 for _ in shape))
    probs, slot0, slot1, ew, bexp, nused = pl.pallas_call(
        _gate_route_body,
        in_specs=[
            full((N_TOK, D_IN)),
            full((N_TOK, D_MKT)),
            full((D_IN, N_EXP)),
            full((D_MKT, N_EXP)),
            full((1, N_EXP)),
        ],
        out_specs=[
            full((N_TOK, N_EXP)),
            full((N_TOK, 1)),
            full((N_TOK, 1)),
            full((N_TOK, TOP_K)),
            full((NB, 1)),
            full((1, 1)),
        ],
        out_shape=[
            jax.ShapeDtypeStruct((N_TOK, N_EXP), jnp.float32),
            jax.ShapeDtypeStruct((N_TOK, 1), jnp.int32),
            jax.ShapeDtypeStruct((N_TOK, 1), jnp.int32),
            jax.ShapeDtypeStruct((N_TOK, TOP_K), jnp.float32),
            jax.ShapeDtypeStruct((NB, 1), jnp.int32),
            jax.ShapeDtypeStruct((1, 1), jnp.int32),
        ],
    )(x, market_status, wgx, wgm, bg2)

    nrows = N_WORKERS * N_CHUNKS
    slots_sc = jnp.stack(
        [slot0.reshape(nrows, SC_CHUNK), slot1.reshape(nrows, SC_CHUNK)], axis=1)
    ws_sc = jnp.stack(
        [ew[:, 0].reshape(nrows, SC_CHUNK), ew[:, 1].reshape(nrows, SC_CHUNK)],
        axis=1)

    xs = _scatter_sc(x, slots_sc)

    grid_spec = pltpu.PrefetchScalarGridSpec(
        num_scalar_prefetch=2,
        grid=(NB,),
        in_specs=[
            pl.BlockSpec((BLK, D_IN),
                         lambda b, be, nu: (jnp.minimum(b, nu[0] - 1), 0)),
            pl.BlockSpec((1, D_IN, D_H), lambda b, be, nu: (be[b], 0, 0)),
            pl.BlockSpec((1, 1, D_H), lambda b, be, nu: (be[b], 0, 0)),
            pl.BlockSpec((1, D_H, D_OUT), lambda b, be, nu: (be[b], 0, 0)),
            pl.BlockSpec((1, 1, D_OUT), lambda b, be, nu: (be[b], 0, 0)),
        ],
        out_specs=pl.BlockSpec((BLK, D_OUT), lambda b, be, nu: (b, 0)),
    )
    y = pl.pallas_call(
        _ffn_body,
        grid_spec=grid_spec,
        out_shape=jax.ShapeDtypeStruct((NSLOT, D_OUT), jnp.float32),
    )(bexp.reshape(NB), nused.reshape(1), xs, W1,
      b1.reshape(N_EXP, 1, D_H), W2, b2.reshape(N_EXP, 1, D_OUT))

    s0 = slot0.reshape(N_TOK)
    s1 = slot1.reshape(N_TOK)
    out = ew[:, 0:1] * y[s0] + ew[:, 1:2] * y[s1]
    return out, probs
